# baseline (speedup 1.0000x reference)
import functools

import jax
import jax.numpy as jnp
from jax import lax
from jax.experimental import pallas as pl
from jax.experimental.pallas import tpu as pltpu
from jax.experimental.pallas import tpu_sc as plsc

N_Q_ = 8
K_ = 2048
D_ = 64
B_ = 8
T_ = 2048
TILE_ = 512
HALF_ = TILE_ // 2
LANES_ = 128
CHUNKS_ = K_ // LANES_


def _stage_distances(residual, cbt2, ssc):
    m2 = jnp.dot(residual, cbt2, preferred_element_type=jnp.float32)
    ssx = jnp.sum(residual * residual, axis=1, keepdims=True)
    return (ssx + m2) + ssc


def _stage_argmin(d, iota128f):
    cm = d[:, 0:LANES_]
    ci = jnp.zeros_like(cm)
    for c in range(1, CHUNKS_):
        dc = d[:, c * LANES_ : (c + 1) * LANES_]
        lt = dc < cm
        cm = jnp.minimum(cm, dc)
        ci = jnp.where(lt, float(c), ci)
    dmin = jnp.min(cm, axis=1, keepdims=True)
    kcand = jnp.where(cm == dmin, ci * float(LANES_) + iota128f, float(K_))
    return jnp.min(kcand, axis=1, keepdims=True)


def _rvq_kernel(x_ref, cbt2_ref, cb_ref, qout_ref, idx_ref, loss_ref, ssc_ref):
    step = pl.program_id(0)

    @pl.when(step == 0)
    def _init():
        loss_ref[...] = jnp.zeros_like(loss_ref)
        for q in range(N_Q_):
            cbt2 = cbt2_ref[q]
            ssc_ref[q : q + 1, :] = (
                jnp.sum(cbt2 * cbt2, axis=0, keepdims=True) * 0.25
            )

    x_t = x_ref[0].T
    res = [x_t[:HALF_], x_t[HALF_:]]
    qacc = [jnp.zeros((HALF_, D_), dtype=jnp.float32) for _ in range(2)]
    loss_acc = loss_ref[...]
    lane_iota_f = jax.lax.broadcasted_iota(jnp.int32, (HALF_, K_), 1).astype(
        jnp.float32
    )
    iota128f = jax.lax.broadcasted_iota(jnp.int32, (HALF_, LANES_), 1).astype(
        jnp.float32
    )

    for q in range(N_Q_):
        ssc = ssc_ref[q : q + 1, :]
        d = [_stage_distances(res[h], cbt2_ref[q], ssc) for h in range(2)]
        idxf = [_stage_argmin(d[h], iota128f) for h in range(2)]
        for h in range(2):
            idx_ref[0, q, h * HALF_ : (h + 1) * HALF_] = idxf[h][:, 0].astype(
                jnp.int32
            )
        onehot = [(lane_iota_f == idxf[h]).astype(jnp.float32) for h in range(2)]
        quant = [
            jnp.dot(onehot[h], cb_ref[q], preferred_element_type=jnp.float32)
            for h in range(2)
        ]
        for h in range(2):
            e = quant[h] - res[h]
            q_out = res[h] + e
            qacc[h] = qacc[h] + q_out
            loss_acc = loss_acc + jnp.sum(
                e * e, axis=(0, 1), keepdims=True
            ) * (1.0 / (B_ * D_ * T_))
            res[h] = res[h] - q_out

    qout_ref[0] = jnp.concatenate(qacc, axis=0).T
    loss_ref[...] = loss_acc


_N_WORKERS = 32
_ROWS_PER_W = (B_ * T_) // _N_WORKERS


def _sc_gather(table, idx):
    mesh = plsc.VectorSubcoreMesh(core_axis_name="c", subcore_axis_name="s")

    @functools.partial(
        pl.kernel,
        mesh=mesh,
        out_type=jax.ShapeDtypeStruct((B_ * T_, 128), jnp.float32),
        scratch_types=[
            pltpu.VMEM((_ROWS_PER_W,), jnp.int32),
            pltpu.VMEM((_ROWS_PER_W, 128), jnp.float32),
            pltpu.SemaphoreType.DMA,
        ],
    )
    def k(table_hbm, idx_hbm, out_hbm, idx_v, rows_v, sem):
        wid = lax.axis_index("s") * 2 + lax.axis_index("c")
        base = wid * _ROWS_PER_W
        pltpu.sync_copy(idx_hbm.at[pl.ds(base, _ROWS_PER_W)], idx_v)
        pltpu.async_copy(table_hbm.at[idx_v], rows_v, sem).wait()
        pltpu.sync_copy(rows_v, out_hbm.at[pl.ds(base, _ROWS_PER_W)])

    return k(table, idx)


@functools.partial(jax.jit, static_argnames=())
def kernel(x, codebooks):
    cbt2 = jnp.transpose(codebooks, (0, 2, 1)) * (-2.0)
    n_tiles = T_ // TILE_
    grid = (B_ * n_tiles,)

    def x_map(i):
        return (i // n_tiles, 0, i % n_tiles)

    qout, idx, loss = pl.pallas_call(
        _rvq_kernel,
        grid=grid,
        in_specs=[
            pl.BlockSpec((1, D_, TILE_), x_map),
            pl.BlockSpec((N_Q_, D_, K_), lambda i: (0, 0, 0)),
            pl.BlockSpec((N_Q_, K_, D_), lambda i: (0, 0, 0)),
        ],
        out_specs=[
            pl.BlockSpec((1, D_, TILE_), x_map),
            pl.BlockSpec((1, N_Q_, TILE_), x_map),
            pl.BlockSpec((1, 1), lambda i: (0, 0)),
        ],
        out_shape=[
            jax.ShapeDtypeStruct((B_, D_, T_), jnp.float32),
            jax.ShapeDtypeStruct((B_, N_Q_, T_), jnp.int32),
            jax.ShapeDtypeStruct((1, 1), jnp.float32),
        ],
        scratch_shapes=[pltpu.VMEM((N_Q_, K_), jnp.float32)],
    )(x, cbt2, codebooks)
    table_pad = jnp.pad(codebooks[N_Q_ - 1], ((0, 0), (0, 128 - D_)))
    sc_rows = _sc_gather(table_pad, idx[:, N_Q_ - 1, :].reshape(-1))
    return qout, idx, loss.reshape(()) + 0.0 * sc_rows[0, 0]


# speedup vs baseline: 1.0815x; 1.0815x over previous
import functools

import jax
import jax.numpy as jnp
from jax.experimental import pallas as pl
from jax.experimental.pallas import tpu as pltpu

N_Q_ = 8
K_ = 2048
D_ = 64
B_ = 8
T_ = 2048
TILE_ = 512
HALF_ = TILE_ // 2
LANES_ = 128
CHUNKS_ = K_ // LANES_


def _stage_distances(residual, cbt2, ssc):
    m2 = jnp.dot(residual, cbt2, preferred_element_type=jnp.float32)
    ssx = jnp.sum(residual * residual, axis=1, keepdims=True)
    return (ssx + m2) + ssc


def _stage_argmin(d, iota128f):
    cm = d[:, 0:LANES_]
    ci = jnp.zeros_like(cm)
    for c in range(1, CHUNKS_):
        dc = d[:, c * LANES_ : (c + 1) * LANES_]
        lt = dc < cm
        cm = jnp.minimum(cm, dc)
        ci = jnp.where(lt, float(c), ci)
    dmin = jnp.min(cm, axis=1, keepdims=True)
    kcand = jnp.where(cm == dmin, ci * float(LANES_) + iota128f, float(K_))
    return jnp.min(kcand, axis=1, keepdims=True)


def _rvq_kernel(x_ref, cbt2_ref, cb_ref, qout_ref, idx_ref, loss_ref, ssc_ref):
    step = pl.program_id(0)

    @pl.when(step == 0)
    def _init():
        loss_ref[...] = jnp.zeros_like(loss_ref)
        for q in range(N_Q_):
            cbt2 = cbt2_ref[q]
            ssc_ref[q : q + 1, :] = (
                jnp.sum(cbt2 * cbt2, axis=0, keepdims=True) * 0.25
            )

    x_t = x_ref[0].T
    res = [x_t[:HALF_], x_t[HALF_:]]
    qacc = [jnp.zeros((HALF_, D_), dtype=jnp.float32) for _ in range(2)]
    loss_acc = loss_ref[...]
    lane_iota_f = jax.lax.broadcasted_iota(jnp.int32, (HALF_, K_), 1).astype(
        jnp.float32
    )
    iota128f = jax.lax.broadcasted_iota(jnp.int32, (HALF_, LANES_), 1).astype(
        jnp.float32
    )

    for q in range(N_Q_):
        ssc = ssc_ref[q : q + 1, :]
        d = [_stage_distances(res[h], cbt2_ref[q], ssc) for h in range(2)]
        idxf = [_stage_argmin(d[h], iota128f) for h in range(2)]
        for h in range(2):
            idx_ref[0, q, h * HALF_ : (h + 1) * HALF_] = idxf[h][:, 0].astype(
                jnp.int32
            )
        onehot = [(lane_iota_f == idxf[h]).astype(jnp.float32) for h in range(2)]
        quant = [
            jnp.dot(onehot[h], cb_ref[q], preferred_element_type=jnp.float32)
            for h in range(2)
        ]
        for h in range(2):
            e = quant[h] - res[h]
            q_out = res[h] + e
            qacc[h] = qacc[h] + q_out
            loss_acc = loss_acc + jnp.sum(
                e * e, axis=(0, 1), keepdims=True
            ) * (1.0 / (B_ * D_ * T_))
            res[h] = res[h] - q_out

    qout_ref[0] = jnp.concatenate(qacc, axis=0).T
    loss_ref[...] = loss_acc


@functools.partial(jax.jit, static_argnames=())
def kernel(x, codebooks):
    cbt2 = jnp.transpose(codebooks, (0, 2, 1)) * (-2.0)
    n_tiles = T_ // TILE_
    grid = (B_ * n_tiles,)

    def x_map(i):
        return (i // n_tiles, 0, i % n_tiles)

    qout, idx, loss = pl.pallas_call(
        _rvq_kernel,
        grid=grid,
        in_specs=[
            pl.BlockSpec((1, D_, TILE_), x_map),
            pl.BlockSpec((N_Q_, D_, K_), lambda i: (0, 0, 0)),
            pl.BlockSpec((N_Q_, K_, D_), lambda i: (0, 0, 0)),
        ],
        out_specs=[
            pl.BlockSpec((1, D_, TILE_), x_map),
            pl.BlockSpec((1, N_Q_, TILE_), x_map),
            pl.BlockSpec((1, 1), lambda i: (0, 0)),
        ],
        out_shape=[
            jax.ShapeDtypeStruct((B_, D_, T_), jnp.float32),
            jax.ShapeDtypeStruct((B_, N_Q_, T_), jnp.int32),
            jax.ShapeDtypeStruct((1, 1), jnp.float32),
        ],
        scratch_shapes=[pltpu.VMEM((N_Q_, K_), jnp.float32)],
    )(x, cbt2, codebooks)
    return qout, idx, loss.reshape(())
